# baseline (speedup 1.0000x reference)
import functools

import jax
import jax.numpy as jnp
from jax import lax
from jax.experimental import pallas as pl
from jax.experimental.pallas import tpu as pltpu
from jax.experimental.pallas import tpu_sc as plsc

_NC = 2
_NS = 16
_NW = _NC * _NS
_L = 16

_M = 1000000
_K = 64
_B = 16384
_W = 512
_RANGE = 31232
_NCH = _RANGE // _W
_TAIL0 = _NW * _RANGE
_TAILB = _TAIL0 + _W
_NB = 64
_NRING = 8
_GROWS = _B + _L
_DUMMY = _B


def _mesh():
    return plsc.VectorSubcoreMesh(core_axis_name="c", subcore_axis_name="s")


def _make_scan_kernel():
    @functools.partial(
        pl.kernel,
        mesh=_mesh(),
        out_type=(
            jax.ShapeDtypeStruct((_GROWS, 128), jnp.float32),
            jax.ShapeDtypeStruct((_GROWS, 128), jnp.float32),
        ),
        scratch_types=[
            pltpu.VMEM((64, _W), jnp.float32),
            pltpu.VMEM((64, _W), jnp.float32),
            pltpu.VMEM((_B,), jnp.int32),
            pltpu.VMEM((_B,), jnp.int32),
            pltpu.VMEM((_NB,), jnp.int32),
            pltpu.VMEM((_NRING, _L, 128), jnp.float32),
            pltpu.VMEM((64, _M - _TAILB), jnp.float32),
            pltpu.SemaphoreType.DMA,
            pltpu.SemaphoreType.DMA,
            pltpu.SemaphoreType.DMA((_NRING,)),
        ],
        compiler_params=pltpu.CompilerParams(needs_layout_passes=False),
    )
    def scan_kernel(su_hbm, sj_hbm, sv_hbm, sw_hbm, bu_hbm, bv_hbm,
                    pt_hbm, qt_hbm, pt_tail, qt_tail,
                    pg_hbm, qg_hbm,
                    buf0, buf1, ids_v, pos_v, bnd_v, stage, tbuf,
                    sem0, sem1, rsem):
        wid = lax.axis_index("s") * _NC + lax.axis_index("c")
        rlo = wid * _RANGE
        lanes = lax.iota(jnp.int32, 16)

        def fire(tab_hbm, coff, buf, sem):
            coff = pl.multiple_of(coff, 128)
            pltpu.async_copy(tab_hbm.at[:, pl.ds(coff, _W)], buf, sem)

        def wait(tab_hbm, buf, sem):
            pltpu.make_async_copy(tab_hbm.at[:, pl.ds(0, _W)], buf, sem).wait()

        def ring_wait(slot, gout_hbm):
            pltpu.make_async_copy(
                gout_hbm.at[pl.ds(0, _L), :], stage.at[slot], rsem.at[slot]).wait()

        def bnd(i):
            b = plsc.load_gather(bnd_v, [jnp.full((16,), 0, jnp.int32) + i])
            return b[0]

        def process_chunk(ci, coff, buf, gout_hbm, gc):
            s = bnd(ci)
            e = bnd(ci + 1)

            def group_body(g, gc):
                slot = lax.rem(gc, _NRING)

                @pl.when(gc >= _NRING)
                def _():
                    ring_wait(slot, gout_hbm)

                p16 = s + g * _L + lanes
                valid = p16 < e
                p16 = jnp.where(valid, p16, s)
                u16 = plsc.load_gather(ids_v, [p16])
                ul = jnp.where(valid, u16 - coff, 0)
                jv = jnp.where(valid, plsc.load_gather(pos_v, [p16]), _DUMMY)
                sv = jnp.full((16,), 0, jnp.int32) + slot
                for k in range(_K):
                    kv = jnp.full((16,), k, jnp.int32)
                    vk = plsc.load_gather(buf, [kv, ul])
                    plsc.store_scatter(stage, [sv, lanes, kv], vk)
                pltpu.async_copy(stage.at[slot], gout_hbm.at[jv], rsem.at[slot])
                return gc + 1

            return lax.fori_loop(0, (e - s + _L - 1) // _L, group_body, gc)

        def scan_table(sids_hbm, spos_hbm, bounds_hbm, tab_hbm, tail_hbm,
                       gout_hbm, gc):
            pltpu.sync_copy(sids_hbm, ids_v)
            pltpu.sync_copy(spos_hbm, pos_v)
            pltpu.sync_copy(bounds_hbm.at[wid], bnd_v)
            fire(tab_hbm, rlo, buf0, sem0)
            fire(tab_hbm, rlo + _W, buf1, sem1)

            def pair_body(i, gc):
                for phase, buf, sem in ((0, buf0, sem0), (1, buf1, sem1)):
                    ci = 2 * i + phase
                    wait(tab_hbm, buf, sem)
                    gc = process_chunk(ci, rlo + ci * _W, buf, gout_hbm, gc)
                    nxt = ci + 2

                    @pl.when(nxt < _NCH)
                    def _():
                        fire(tab_hbm, rlo + nxt * _W, buf, sem)
                return gc

            gc = lax.fori_loop(0, _NCH // 2, pair_body, gc)
            wait(tab_hbm, buf0, sem0)
            gc = process_chunk(_NCH - 1, rlo + (_NCH - 1) * _W, buf0,
                               gout_hbm, gc)

            def tail_work(gc):
                fire(tab_hbm, _TAIL0, buf0, sem0)
                wait(tab_hbm, buf0, sem0)
                gc = process_chunk(_NCH, _TAIL0, buf0, gout_hbm, gc)
                pltpu.sync_copy(tail_hbm, tbuf)
                return process_chunk(_NCH + 1, _TAILB, tbuf, gout_hbm, gc)

            gc = lax.cond(wid == _NW - 1, tail_work, lambda gc: gc, gc)

            for t in range(_NRING):
                @pl.when(gc > t)
                def _():
                    ring_wait(t, gout_hbm)
            return jnp.int32(0)

        gc = scan_table(su_hbm, sj_hbm, bu_hbm, pt_hbm, pt_tail, pg_hbm,
                        jnp.int32(0))
        scan_table(sv_hbm, sw_hbm, bv_hbm, qt_hbm, qt_tail, qg_hbm, gc)

    return scan_kernel


def _make_dot_kernel():
    b_per_w = _B // _NW
    step = 128

    @functools.partial(
        pl.kernel,
        mesh=_mesh(),
        out_type=jax.ShapeDtypeStruct((_B,), jnp.float32),
        scratch_types=[
            pltpu.VMEM((2, step, 128), jnp.float32),
            pltpu.VMEM((2, step, 128), jnp.float32),
            pltpu.VMEM((b_per_w,), jnp.float32),
            pltpu.SemaphoreType.DMA,
            pltpu.SemaphoreType.DMA,
        ],
        compiler_params=pltpu.CompilerParams(needs_layout_passes=False),
    )
    def dot_kernel(pg_hbm, qg_hbm, out_hbm, pbuf, qbuf, out_v, sem0, sem1):
        wid = lax.axis_index("s") * _NC + lax.axis_index("c")
        base = wid * b_per_w
        lanes = lax.iota(jnp.int32, 16)
        nsteps = b_per_w // step
        sems = (sem0, sem1)

        def fire(h, slot):
            off = pl.multiple_of(base + h * step, 8)
            pltpu.async_copy(pg_hbm.at[pl.ds(off, step), :], pbuf.at[slot], sems[slot])
            pltpu.async_copy(qg_hbm.at[pl.ds(off, step), :], qbuf.at[slot], sems[slot])

        def wait(slot):
            pltpu.make_async_copy(pg_hbm.at[pl.ds(0, step), :], pbuf.at[slot], sems[slot]).wait()
            pltpu.make_async_copy(qg_hbm.at[pl.ds(0, step), :], qbuf.at[slot], sems[slot]).wait()

        fire(0, 0)
        fire(1, 1)
        for h in range(nsteps):
            slot = h % 2
            wait(slot)

            def group_body(g, carry):
                rloc = g * _L + lanes
                acc = jnp.zeros((16,), jnp.float32)
                for k in range(_K):
                    kv = jnp.full((16,), k, jnp.int32)
                    pv = plsc.load_gather(pbuf, [jnp.full((16,), slot, jnp.int32), rloc, kv])
                    qv = plsc.load_gather(qbuf, [jnp.full((16,), slot, jnp.int32), rloc, kv])
                    acc = acc + pv * qv
                plsc.store_scatter(out_v, [h * step + rloc], acc)
                return carry

            lax.fori_loop(0, step // _L, group_body, 0)
            if h + 2 < nsteps:
                fire(h + 2, slot)

        pltpu.sync_copy(out_v, out_hbm.at[pl.ds(base, b_per_w)])

    return dot_kernel


def _bounds(sorted_ids):
    w = jnp.arange(_NW, dtype=jnp.int32)[:, None]
    ci = jnp.arange(_NB, dtype=jnp.int32)[None, :]
    col = w * _RANGE + jnp.minimum(ci, _NCH) * _W
    tail_col = jnp.where(ci == _NCH, _TAIL0,
                         jnp.where(ci == _NCH + 1, _TAILB, _M))
    col = jnp.where(ci <= _NCH, col, jnp.where(w == _NW - 1, tail_col,
                                               (w + 1) * _RANGE))
    return jnp.searchsorted(sorted_ids, col.reshape(-1),
                            side="left").astype(jnp.int32).reshape(_NW, _NB)


def kernel(user_ids, item_ids, P, Q):
    uid = user_ids.astype(jnp.int32)
    iid = item_ids.astype(jnp.int32)
    iota = jnp.arange(_B, dtype=jnp.int32)
    su, sj = lax.sort_key_val(uid, iota)
    sv, sw = lax.sort_key_val(iid, iota)
    bu = _bounds(su)
    bv = _bounds(sv)
    pt, qt = P.T, Q.T
    pg, qg = _make_scan_kernel()(su, sj, sv, sw, bu, bv, pt, qt,
                                 pt[:, _TAILB:], qt[:, _TAILB:])
    return _make_dot_kernel()(pg, qg)


# speedup vs baseline: 1.0026x; 1.0026x over previous
import functools

import jax
import jax.numpy as jnp
from jax import lax
from jax.experimental import pallas as pl
from jax.experimental.pallas import tpu as pltpu
from jax.experimental.pallas import tpu_sc as plsc

_NC = 2
_NS = 16
_NW = _NC * _NS
_L = 16

_M = 1000000
_K = 64
_B = 16384
_W = 512
_RANGE = 31232
_NCH = _RANGE // _W
_TAIL0 = _NW * _RANGE
_TAILB = _TAIL0 + _W
_NB = 64
_NRING = 8
_GROWS = _B + _L
_DUMMY = _B


def _mesh():
    return plsc.VectorSubcoreMesh(core_axis_name="c", subcore_axis_name="s")


def _make_scan_kernel():
    @functools.partial(
        pl.kernel,
        mesh=_mesh(),
        out_type=(
            jax.ShapeDtypeStruct((_GROWS, 128), jnp.float32),
            jax.ShapeDtypeStruct((_GROWS, 128), jnp.float32),
        ),
        scratch_types=[
            pltpu.VMEM((64, _W), jnp.float32),
            pltpu.VMEM((64, _W), jnp.float32),
            pltpu.VMEM((_B,), jnp.int32),
            pltpu.VMEM((_B,), jnp.int32),
            pltpu.VMEM((_NB,), jnp.int32),
            pltpu.SMEM((_NB,), jnp.int32),
            pltpu.VMEM((_NRING, _L, 128), jnp.float32),
            pltpu.VMEM((64, _M - _TAILB), jnp.float32),
            pltpu.SemaphoreType.DMA,
            pltpu.SemaphoreType.DMA,
            pltpu.SemaphoreType.DMA((_NRING,)),
        ],
        compiler_params=pltpu.CompilerParams(needs_layout_passes=False),
    )
    def scan_kernel(su_hbm, sj_hbm, sv_hbm, sw_hbm, bu_hbm, bv_hbm,
                    pt_hbm, qt_hbm, pt_tail, qt_tail,
                    pg_hbm, qg_hbm,
                    buf0, buf1, ids_v, pos_v, bnd_vm, bnd_v, stage, tbuf,
                    sem0, sem1, rsem):
        wid = lax.axis_index("s") * _NC + lax.axis_index("c")
        rlo = wid * _RANGE
        lanes = lax.iota(jnp.int32, 16)

        def fire(tab_hbm, coff, buf, sem):
            coff = pl.multiple_of(coff, 128)
            pltpu.async_copy(tab_hbm.at[:, pl.ds(coff, _W)], buf, sem)

        def wait(tab_hbm, buf, sem):
            pltpu.make_async_copy(tab_hbm.at[:, pl.ds(0, _W)], buf, sem).wait()

        def ring_wait(slot, gout_hbm):
            pltpu.make_async_copy(
                gout_hbm.at[pl.ds(0, _L), :], stage.at[slot], rsem.at[slot]).wait()

        def bnd(i):
            return bnd_v[i]

        def process_chunk(ci, coff, buf, gout_hbm, gc):
            s = bnd(ci)
            e = bnd(ci + 1)

            def group_body(g, gc):
                slot = lax.rem(gc, _NRING)

                @pl.when(gc >= _NRING)
                def _():
                    ring_wait(slot, gout_hbm)

                p16 = s + g * _L + lanes
                valid = p16 < e
                p16 = jnp.where(valid, p16, s)
                u16 = plsc.load_gather(ids_v, [p16])
                ul = jnp.where(valid, u16 - coff, 0)
                jv = jnp.where(valid, plsc.load_gather(pos_v, [p16]), _DUMMY)
                sv = jnp.full((16,), 0, jnp.int32) + slot
                for k in range(_K):
                    kv = jnp.full((16,), k, jnp.int32)
                    vk = plsc.load_gather(buf, [kv, ul])
                    plsc.store_scatter(stage, [sv, lanes, kv], vk)
                pltpu.async_copy(stage.at[slot], gout_hbm.at[jv], rsem.at[slot])
                return gc + 1

            return lax.fori_loop(0, (e - s + _L - 1) // _L, group_body, gc)

        def scan_table(sids_hbm, spos_hbm, bounds_hbm, tab_hbm, tail_hbm,
                       gout_hbm, gc):
            pltpu.sync_copy(sids_hbm, ids_v)
            pltpu.sync_copy(spos_hbm, pos_v)
            pltpu.sync_copy(bounds_hbm.at[wid], bnd_vm)

            def cp_bound(i, c):
                v = plsc.load_gather(bnd_vm, [jnp.full((16,), 0, jnp.int32) + i])
                bnd_v[i] = v[0]
                return c

            lax.fori_loop(0, _NB, cp_bound, 0)
            fire(tab_hbm, rlo, buf0, sem0)
            fire(tab_hbm, rlo + _W, buf1, sem1)

            def pair_body(i, gc):
                for phase, buf, sem in ((0, buf0, sem0), (1, buf1, sem1)):
                    ci = 2 * i + phase
                    wait(tab_hbm, buf, sem)
                    gc = process_chunk(ci, rlo + ci * _W, buf, gout_hbm, gc)
                    nxt = ci + 2

                    @pl.when(nxt < _NCH)
                    def _():
                        fire(tab_hbm, rlo + nxt * _W, buf, sem)
                return gc

            gc = lax.fori_loop(0, _NCH // 2, pair_body, gc)
            wait(tab_hbm, buf0, sem0)
            gc = process_chunk(_NCH - 1, rlo + (_NCH - 1) * _W, buf0,
                               gout_hbm, gc)

            def tail_work(gc):
                fire(tab_hbm, _TAIL0, buf0, sem0)
                wait(tab_hbm, buf0, sem0)
                gc = process_chunk(_NCH, _TAIL0, buf0, gout_hbm, gc)
                pltpu.sync_copy(tail_hbm, tbuf)
                return process_chunk(_NCH + 1, _TAILB, tbuf, gout_hbm, gc)

            gc = lax.cond(wid == _NW - 1, tail_work, lambda gc: gc, gc)

            for t in range(_NRING):
                @pl.when(gc > t)
                def _():
                    ring_wait(t, gout_hbm)
            return jnp.int32(0)

        gc = scan_table(su_hbm, sj_hbm, bu_hbm, pt_hbm, pt_tail, pg_hbm,
                        jnp.int32(0))
        scan_table(sv_hbm, sw_hbm, bv_hbm, qt_hbm, qt_tail, qg_hbm, gc)

    return scan_kernel


def _make_dot_kernel():
    b_per_w = _B // _NW
    step = 128

    @functools.partial(
        pl.kernel,
        mesh=_mesh(),
        out_type=jax.ShapeDtypeStruct((_B,), jnp.float32),
        scratch_types=[
            pltpu.VMEM((2, step, 128), jnp.float32),
            pltpu.VMEM((2, step, 128), jnp.float32),
            pltpu.VMEM((b_per_w,), jnp.float32),
            pltpu.SemaphoreType.DMA,
            pltpu.SemaphoreType.DMA,
        ],
        compiler_params=pltpu.CompilerParams(needs_layout_passes=False),
    )
    def dot_kernel(pg_hbm, qg_hbm, out_hbm, pbuf, qbuf, out_v, sem0, sem1):
        wid = lax.axis_index("s") * _NC + lax.axis_index("c")
        base = wid * b_per_w
        lanes = lax.iota(jnp.int32, 16)
        nsteps = b_per_w // step
        sems = (sem0, sem1)

        def fire(h, slot):
            off = pl.multiple_of(base + h * step, 8)
            pltpu.async_copy(pg_hbm.at[pl.ds(off, step), :], pbuf.at[slot], sems[slot])
            pltpu.async_copy(qg_hbm.at[pl.ds(off, step), :], qbuf.at[slot], sems[slot])

        def wait(slot):
            pltpu.make_async_copy(pg_hbm.at[pl.ds(0, step), :], pbuf.at[slot], sems[slot]).wait()
            pltpu.make_async_copy(qg_hbm.at[pl.ds(0, step), :], qbuf.at[slot], sems[slot]).wait()

        fire(0, 0)
        fire(1, 1)
        for h in range(nsteps):
            slot = h % 2
            wait(slot)

            def group_body(g, carry):
                rloc = g * _L + lanes
                acc = jnp.zeros((16,), jnp.float32)
                for k in range(_K):
                    kv = jnp.full((16,), k, jnp.int32)
                    pv = plsc.load_gather(pbuf, [jnp.full((16,), slot, jnp.int32), rloc, kv])
                    qv = plsc.load_gather(qbuf, [jnp.full((16,), slot, jnp.int32), rloc, kv])
                    acc = acc + pv * qv
                plsc.store_scatter(out_v, [h * step + rloc], acc)
                return carry

            lax.fori_loop(0, step // _L, group_body, 0)
            if h + 2 < nsteps:
                fire(h + 2, slot)

        pltpu.sync_copy(out_v, out_hbm.at[pl.ds(base, b_per_w)])

    return dot_kernel


def _bounds(sorted_ids):
    w = jnp.arange(_NW, dtype=jnp.int32)[:, None]
    ci = jnp.arange(_NB, dtype=jnp.int32)[None, :]
    col = w * _RANGE + jnp.minimum(ci, _NCH) * _W
    tail_col = jnp.where(ci == _NCH, _TAIL0,
                         jnp.where(ci == _NCH + 1, _TAILB, _M))
    col = jnp.where(ci <= _NCH, col, jnp.where(w == _NW - 1, tail_col,
                                               (w + 1) * _RANGE))
    return jnp.searchsorted(sorted_ids, col.reshape(-1),
                            side="left").astype(jnp.int32).reshape(_NW, _NB)


def kernel(user_ids, item_ids, P, Q):
    uid = user_ids.astype(jnp.int32)
    iid = item_ids.astype(jnp.int32)
    iota = jnp.arange(_B, dtype=jnp.int32)
    su, sj = lax.sort_key_val(uid, iota)
    sv, sw = lax.sort_key_val(iid, iota)
    bu = _bounds(su)
    bv = _bounds(sv)
    pt, qt = P.T, Q.T
    pg, qg = _make_scan_kernel()(su, sj, sv, sw, bu, bv, pt, qt,
                                 pt[:, _TAILB:], qt[:, _TAILB:])
    return _make_dot_kernel()(pg, qg)


# speedup vs baseline: 1.0027x; 1.0001x over previous
import functools

import jax
import jax.numpy as jnp
from jax import lax
from jax.experimental import pallas as pl
from jax.experimental.pallas import tpu as pltpu
from jax.experimental.pallas import tpu_sc as plsc

_NC = 2
_NS = 16
_NW = _NC * _NS
_L = 16

_M = 1000000
_K = 64
_B = 16384
_W = 512
_RANGE = 31232
_NCH = _RANGE // _W
_TAIL0 = _NW * _RANGE
_TAILB = _TAIL0 + _W
_NB = 64
_NRING = 8
_GROWS = _B + _L
_DUMMY = _B


def _mesh():
    return plsc.VectorSubcoreMesh(core_axis_name="c", subcore_axis_name="s")


def _make_scan_kernel():
    @functools.partial(
        pl.kernel,
        mesh=_mesh(),
        out_type=(
            jax.ShapeDtypeStruct((_GROWS, 128), jnp.float32),
            jax.ShapeDtypeStruct((_GROWS, 128), jnp.float32),
        ),
        scratch_types=[
            pltpu.VMEM((64, _W), jnp.float32),
            pltpu.VMEM((64, _W), jnp.float32),
            pltpu.VMEM((_B,), jnp.int32),
            pltpu.VMEM((_B,), jnp.int32),
            pltpu.VMEM((_NB,), jnp.int32),
            pltpu.SMEM((_NB,), jnp.int32),
            pltpu.VMEM((_NRING, _L, 128), jnp.float32),
            pltpu.VMEM((64, _M - _TAILB), jnp.float32),
            pltpu.SemaphoreType.DMA,
            pltpu.SemaphoreType.DMA,
            pltpu.SemaphoreType.DMA((_NRING,)),
        ],
        compiler_params=pltpu.CompilerParams(needs_layout_passes=False),
    )
    def scan_kernel(su_hbm, sj_hbm, sv_hbm, sw_hbm, bu_hbm, bv_hbm,
                    pt_hbm, qt_hbm, pt_tail, qt_tail,
                    pg_hbm, qg_hbm,
                    buf0, buf1, ids_v, pos_v, bnd_vm, bnd_v, stage, tbuf,
                    sem0, sem1, rsem):
        wid = lax.axis_index("s") * _NC + lax.axis_index("c")
        rlo = wid * _RANGE
        lanes = lax.iota(jnp.int32, 16)

        def fire(tab_hbm, coff, buf, sem):
            coff = pl.multiple_of(coff, 128)
            pltpu.async_copy(tab_hbm.at[:, pl.ds(coff, _W)], buf, sem)

        def wait(tab_hbm, buf, sem):
            pltpu.make_async_copy(tab_hbm.at[:, pl.ds(0, _W)], buf, sem).wait()

        def ring_wait(slot, gout_hbm):
            pltpu.make_async_copy(
                gout_hbm.at[pl.ds(0, _L), :], stage.at[slot], rsem.at[slot]).wait()

        def bnd(i):
            return bnd_v[i]

        def process_chunk(ci, coff, buf, gout_hbm, gc):
            s = bnd(ci)
            e = bnd(ci + 1)

            def group_body(g, gc):
                slot = lax.rem(gc, _NRING)

                @pl.when(gc >= _NRING)
                def _():
                    ring_wait(slot, gout_hbm)

                p16 = s + g * _L + lanes
                valid = p16 < e
                p16 = jnp.where(valid, p16, s)
                u16 = plsc.load_gather(ids_v, [p16])
                ul = jnp.where(valid, u16 - coff, 0)
                jv = jnp.where(valid, plsc.load_gather(pos_v, [p16]), _DUMMY)
                sv = jnp.full((16,), 0, jnp.int32) + slot

                def kbody(k, c):
                    kv = jnp.full((16,), 0, jnp.int32) + k
                    vk = plsc.load_gather(buf, [kv, ul])
                    plsc.store_scatter(stage, [sv, lanes, kv], vk)
                    return c

                lax.fori_loop(0, _K, kbody, 0)
                pltpu.async_copy(stage.at[slot], gout_hbm.at[jv], rsem.at[slot])
                return gc + 1

            return lax.fori_loop(0, (e - s + _L - 1) // _L, group_body, gc)

        def scan_table(sids_hbm, spos_hbm, bounds_hbm, tab_hbm, tail_hbm,
                       gout_hbm, gc):
            pltpu.sync_copy(sids_hbm, ids_v)
            pltpu.sync_copy(spos_hbm, pos_v)
            pltpu.sync_copy(bounds_hbm.at[wid], bnd_vm)

            def cp_bound(i, c):
                v = plsc.load_gather(bnd_vm, [jnp.full((16,), 0, jnp.int32) + i])
                bnd_v[i] = v[0]
                return c

            lax.fori_loop(0, _NB, cp_bound, 0)
            fire(tab_hbm, rlo, buf0, sem0)
            fire(tab_hbm, rlo + _W, buf1, sem1)

            def pair_body(i, gc):
                for phase, buf, sem in ((0, buf0, sem0), (1, buf1, sem1)):
                    ci = 2 * i + phase
                    wait(tab_hbm, buf, sem)
                    gc = process_chunk(ci, rlo + ci * _W, buf, gout_hbm, gc)
                    nxt = ci + 2

                    @pl.when(nxt < _NCH)
                    def _():
                        fire(tab_hbm, rlo + nxt * _W, buf, sem)
                return gc

            gc = lax.fori_loop(0, _NCH // 2, pair_body, gc)
            wait(tab_hbm, buf0, sem0)
            gc = process_chunk(_NCH - 1, rlo + (_NCH - 1) * _W, buf0,
                               gout_hbm, gc)

            def tail_work(gc):
                fire(tab_hbm, _TAIL0, buf0, sem0)
                wait(tab_hbm, buf0, sem0)
                gc = process_chunk(_NCH, _TAIL0, buf0, gout_hbm, gc)
                pltpu.sync_copy(tail_hbm, tbuf)
                return process_chunk(_NCH + 1, _TAILB, tbuf, gout_hbm, gc)

            gc = lax.cond(wid == _NW - 1, tail_work, lambda gc: gc, gc)

            for t in range(_NRING):
                @pl.when(gc > t)
                def _():
                    ring_wait(t, gout_hbm)
            return jnp.int32(0)

        gc = scan_table(su_hbm, sj_hbm, bu_hbm, pt_hbm, pt_tail, pg_hbm,
                        jnp.int32(0))
        scan_table(sv_hbm, sw_hbm, bv_hbm, qt_hbm, qt_tail, qg_hbm, gc)

    return scan_kernel


def _make_dot_kernel():
    b_per_w = _B // _NW
    step = 128

    @functools.partial(
        pl.kernel,
        mesh=_mesh(),
        out_type=jax.ShapeDtypeStruct((_B,), jnp.float32),
        scratch_types=[
            pltpu.VMEM((2, step, 128), jnp.float32),
            pltpu.VMEM((2, step, 128), jnp.float32),
            pltpu.VMEM((b_per_w,), jnp.float32),
            pltpu.SemaphoreType.DMA,
            pltpu.SemaphoreType.DMA,
        ],
        compiler_params=pltpu.CompilerParams(needs_layout_passes=False),
    )
    def dot_kernel(pg_hbm, qg_hbm, out_hbm, pbuf, qbuf, out_v, sem0, sem1):
        wid = lax.axis_index("s") * _NC + lax.axis_index("c")
        base = wid * b_per_w
        lanes = lax.iota(jnp.int32, 16)
        nsteps = b_per_w // step
        sems = (sem0, sem1)

        def fire(h, slot):
            off = pl.multiple_of(base + h * step, 8)
            pltpu.async_copy(pg_hbm.at[pl.ds(off, step), :], pbuf.at[slot], sems[slot])
            pltpu.async_copy(qg_hbm.at[pl.ds(off, step), :], qbuf.at[slot], sems[slot])

        def wait(slot):
            pltpu.make_async_copy(pg_hbm.at[pl.ds(0, step), :], pbuf.at[slot], sems[slot]).wait()
            pltpu.make_async_copy(qg_hbm.at[pl.ds(0, step), :], qbuf.at[slot], sems[slot]).wait()

        fire(0, 0)
        fire(1, 1)
        for h in range(nsteps):
            slot = h % 2
            wait(slot)

            def group_body(g, carry):
                rloc = g * _L + lanes
                acc = jnp.zeros((16,), jnp.float32)
                for k in range(_K):
                    kv = jnp.full((16,), k, jnp.int32)
                    pv = plsc.load_gather(pbuf, [jnp.full((16,), slot, jnp.int32), rloc, kv])
                    qv = plsc.load_gather(qbuf, [jnp.full((16,), slot, jnp.int32), rloc, kv])
                    acc = acc + pv * qv
                plsc.store_scatter(out_v, [h * step + rloc], acc)
                return carry

            lax.fori_loop(0, step // _L, group_body, 0)
            if h + 2 < nsteps:
                fire(h + 2, slot)

        pltpu.sync_copy(out_v, out_hbm.at[pl.ds(base, b_per_w)])

    return dot_kernel


def _bounds(sorted_ids):
    w = jnp.arange(_NW, dtype=jnp.int32)[:, None]
    ci = jnp.arange(_NB, dtype=jnp.int32)[None, :]
    col = w * _RANGE + jnp.minimum(ci, _NCH) * _W
    tail_col = jnp.where(ci == _NCH, _TAIL0,
                         jnp.where(ci == _NCH + 1, _TAILB, _M))
    col = jnp.where(ci <= _NCH, col, jnp.where(w == _NW - 1, tail_col,
                                               (w + 1) * _RANGE))
    return jnp.searchsorted(sorted_ids, col.reshape(-1),
                            side="left").astype(jnp.int32).reshape(_NW, _NB)


def kernel(user_ids, item_ids, P, Q):
    uid = user_ids.astype(jnp.int32)
    iid = item_ids.astype(jnp.int32)
    iota = jnp.arange(_B, dtype=jnp.int32)
    su, sj = lax.sort_key_val(uid, iota)
    sv, sw = lax.sort_key_val(iid, iota)
    bu = _bounds(su)
    bv = _bounds(sv)
    pt, qt = P.T, Q.T
    pg, qg = _make_scan_kernel()(su, sj, sv, sw, bu, bv, pt, qt,
                                 pt[:, _TAILB:], qt[:, _TAILB:])
    return _make_dot_kernel()(pg, qg)
